# baseline (speedup 1.0000x reference)
import jax
import jax.numpy as jnp
from jax import lax
from jax.experimental import pallas as pl
from jax.experimental.pallas import tpu as pltpu
from jax.experimental.pallas import tpu_sc as plsc
from jax.experimental.layout import Layout, Format, with_layout_constraint

D_MODEL = 64
SCALE = 8.0
GRP = 128
K = 2
NBUF = 4
NC = 2
NS = 16
NW = NC * NS


def _emb_body(x_hbm, t_hbm, out_hbm, idx_v, bufs, gsem, osem):
    c = lax.axis_index("c")
    s = lax.axis_index("s")
    wid = s * NC + c
    gpw = x_hbm.shape[0] // NW
    nchunks = gpw // K
    grp_base = wid * gpw

    pltpu.sync_copy(x_hbm.at[pl.ds(grp_base, gpw)], idx_v)

    def fire(ch, b):
        for j in range(K):
            pltpu.async_copy(
                t_hbm.at[idx_v.at[ch * K + j]],
                bufs.at[b, j],
                gsem.at[b],
            )

    def wait_gather(b):
        pltpu.make_async_copy(
            out_hbm.at[pl.ds(0, K)], bufs.at[b], gsem.at[b]
        ).wait()

    def fire_out(ch, b):
        pltpu.async_copy(
            bufs.at[b], out_hbm.at[pl.ds(grp_base + ch * K, K)],
            osem.at[b],
        )

    def wait_out(b):
        pltpu.make_async_copy(
            bufs.at[b], out_hbm.at[pl.ds(0, K)], osem.at[b]
        ).wait()

    def scale(b):
        def scale_row(r, _):
            for j in range(K):
                for cix in range(D_MODEL // 16):
                    sl = pl.ds(cix * 16, 16)
                    bufs[b, j, r, sl] = bufs[b, j, r, sl] * SCALE
            return 0

        lax.fori_loop(0, GRP, scale_row, 0, unroll=4)

    for b in range(NBUF - 1):
        fire(b, b)

    def outer(i, carry):
        for b in range(NBUF):
            ch = i * NBUF + b
            nb = (b + NBUF - 1) % NBUF
            nch = ch + NBUF - 1

            @pl.when(jnp.logical_and(nch < nchunks, nch >= NBUF))
            def _():
                wait_out(nb)
                fire(nch, nb)

            @pl.when(jnp.logical_and(nch < nchunks, nch < NBUF))
            def _():
                fire(nch, nb)

            wait_gather(b)
            scale(b)
            fire_out(ch, b)
        return carry

    lax.fori_loop(0, nchunks // NBUF, outer, 0)

    for b in range(NBUF):
        wait_out(b)


def _kernel_impl(x, table, dev):
    orig_shape = x.shape
    b = x.size
    assert b % (NW * GRP * K * NBUF) == 0
    ngroups = b // GRP
    xi = x.reshape(ngroups, GRP).astype(jnp.int32)
    t_lin = table

    out = pl.kernel(
        _emb_body,
        out_type=jax.ShapeDtypeStruct((ngroups, GRP, D_MODEL), jnp.float32),
        mesh=plsc.VectorSubcoreMesh(core_axis_name="c", subcore_axis_name="s"),
        scratch_types=[
            pltpu.VMEM((ngroups // NW, GRP), jnp.int32),
            pltpu.VMEM((NBUF, K, GRP, D_MODEL), jnp.float32),
            pltpu.SemaphoreType.DMA((NBUF,)),
            pltpu.SemaphoreType.DMA((NBUF,)),
        ],
        compiler_params=pltpu.CompilerParams(use_tc_tiling_on_sc=False),
    )(xi, t_lin)
    return out.reshape(*orig_shape, D_MODEL)


_jitted_cache = {}


def _make_jitted(dev):
    fn = _jitted_cache.get(dev)
    if fn is None:
        fmt = Format(
            Layout(major_to_minor=(0, 1, 2), tiling=((8,),)),
            jax.sharding.SingleDeviceSharding(dev),
        )
        import functools
        fn = jax.jit(
            functools.partial(_kernel_impl, dev=dev), out_shardings=fmt
        )
        _jitted_cache[dev] = fn
    return fn


def kernel(x, table):
    try:
        dev = next(iter(table.devices()))
    except Exception:
        try:
            from jax._src import config as _jcfg
            dev = _jcfg.device_context.value.devices.flat[0]
        except Exception:
            dev = jax.devices()[0]
    return _make_jitted(dev)(x, table)
